# baseline (speedup 1.0000x reference)
import jax
import jax.numpy as jnp
from jax import lax
from jax.experimental import pallas as pl
from jax.experimental.pallas import tpu as pltpu
from jax.experimental.pallas import tpu_sc as plsc

N = 10000
E = 320000
D_IN = 128
H = 16
NCORES = 2
NSUB = 16
NW = NCORES * NSUB
EB = 128
ECH = 80
EPT = EB * ECH
RPT = E // NW
NPAD = EPT - RPT
NACC = N + 256
RPS = N // NSUB
ZPS = NACC // NSUB

_mesh = plsc.VectorSubcoreMesh(core_axis_name="c", subcore_axis_name="s",
                               num_cores=NCORES, num_subcores=NSUB)

_sc_params = pltpu.CompilerParams(needs_layout_passes=False,
                                  use_tc_tiling_on_sc=False)


def _sc_deg_body(dst_hbm, deg_hbm, ldeg, dstb, isem):
    c = lax.axis_index("c")
    s = lax.axis_index("s")
    t = c * NSUB + s

    cp = pltpu.async_copy(dst_hbm.at[t], dstb, isem)

    zero16 = jnp.zeros((H,), jnp.float32)

    def _zero(i, _):
        ldeg[pl.ds(i * H, H)] = zero16
        return 0
    lax.fori_loop(0, NACC // H, _zero, 0)

    cp.wait()
    ones16 = jnp.ones((H,), jnp.float32)

    def _hist(r, _):
        for k in range(EB // H):
            idx = dstb[r, pl.ds(k * H, H)]
            plsc.addupdate_scatter(ldeg, [idx], ones16)
        return 0
    lax.fori_loop(0, ECH, _hist, 0)

    pltpu.sync_copy(ldeg.at[pl.ds(0, N)], deg_hbm.at[t])


def _make_sc_deg():
    return pl.kernel(
        _sc_deg_body,
        out_type=jax.ShapeDtypeStruct((NW, N), jnp.float32),
        mesh=_mesh,
        scratch_types=[
            pltpu.VMEM((NACC,), jnp.float32),
            pltpu.VMEM((ECH, EB), jnp.int32),
            pltpu.SemaphoreType.DMA,
        ],
        compiler_params=_sc_params,
    )


GRP = 5
NG = ECH // GRP


def _sc_agg_kernel(g_hbm, src_hbm, dst_hbm, agg_hbm, acc, srcb, dstb,
                   bufs_a, bufs_b, zbuf, gsem_a, gsem_b, ssem_a, ssem_b,
                   isem):
    c = lax.axis_index("c")
    s = lax.axis_index("s")
    t = c * NSUB + s

    cp_src = pltpu.async_copy(src_hbm.at[pl.ds(t * RPT, RPT)],
                              srcb.at[pl.ds(0, RPT)], isem)
    cp_dst = pltpu.async_copy(dst_hbm.at[t], dstb, isem)

    zero16 = jnp.zeros((H,), jnp.float32)

    def _zero(i, _):
        zbuf[i] = zero16
        return 0
    lax.fori_loop(0, ZPS, _zero, 0)
    pltpu.sync_copy(zbuf, acc.at[pl.ds(ZPS * s, ZPS)])

    for i in range(NPAD // H):
        srcb[pl.ds(RPT + i * H, H)] = jnp.zeros((H,), jnp.int32)

    cp_src.wait()
    cp_dst.wait()
    plsc.subcore_barrier()

    def _drain(sem):
        pltpu.make_async_copy(g_hbm.at[pl.ds(0, EB)], bufs_a.at[0], sem).wait()

    def _gathers(grp, bufs, gsem):
        for k in range(GRP):
            j = grp * GRP + k
            pltpu.async_copy(g_hbm.at[srcb.at[pl.ds(j * EB, EB)]],
                             bufs.at[k], gsem)

    def _scatters(grp, bufs, ssem):
        for k in range(GRP):
            pltpu.async_copy(bufs.at[k], acc.at[dstb.at[grp * GRP + k]], ssem,
                             add=True)

    _gathers(0, bufs_a, gsem_a)

    def _step(i, _):
        for _k in range(GRP):
            _drain(gsem_a)
        _scatters(2 * i, bufs_a, ssem_a)

        @pl.when(i > 0)
        def _():
            for _k in range(GRP):
                _drain(ssem_b)
        _gathers(2 * i + 1, bufs_b, gsem_b)

        for _k in range(GRP):
            _drain(gsem_b)
        _scatters(2 * i + 1, bufs_b, ssem_b)
        for _k in range(GRP):
            _drain(ssem_a)
        _gathers(2 * i + 2, bufs_a, gsem_a)
        return 0
    lax.fori_loop(0, (NG - 2) // 2, _step, 0)

    for _k in range(GRP):
        _drain(gsem_a)
    _scatters(NG - 2, bufs_a, ssem_a)
    for _k in range(GRP):
        _drain(ssem_b)
    _gathers(NG - 1, bufs_b, gsem_b)
    for _k in range(GRP):
        _drain(gsem_b)
    _scatters(NG - 1, bufs_b, ssem_b)
    for _k in range(GRP):
        _drain(ssem_a)
    for _k in range(GRP):
        _drain(ssem_b)

    plsc.subcore_barrier()
    pltpu.sync_copy(acc.at[pl.ds(RPS * s, RPS)],
                    agg_hbm.at[c].at[pl.ds(RPS * s, RPS)])


def _make_sc_agg():
    return pl.kernel(
        _sc_agg_kernel,
        out_type=jax.ShapeDtypeStruct((NCORES, N, H), jnp.float32),
        mesh=_mesh,
        scratch_types=[
            pltpu.VMEM_SHARED((NACC, H), jnp.float32),
            pltpu.VMEM((EPT,), jnp.int32),
            pltpu.VMEM((ECH, EB), jnp.int32),
            pltpu.VMEM((GRP, EB, H), jnp.float32),
            pltpu.VMEM((GRP, EB, H), jnp.float32),
            pltpu.VMEM((ZPS, H), jnp.float32),
            pltpu.SemaphoreType.DMA,
            pltpu.SemaphoreType.DMA,
            pltpu.SemaphoreType.DMA,
            pltpu.SemaphoreType.DMA,
            pltpu.SemaphoreType.DMA,
        ],
        compiler_params=_sc_params,
    )


_RB = 5000
_GRID = N // _RB


def _dinv_of(degt_block):
    return lax.rsqrt(jnp.sum(degt_block, axis=1, keepdims=True) + 1.0)


def _tc1_body(x_ref, w_ref, degt_ref, g_ref):
    h = jnp.dot(x_ref[...], w_ref[...], preferred_element_type=jnp.float32)
    g_ref[...] = h * _dinv_of(degt_ref[...])


def _tc_mid_body(a0_ref, a1_ref, g_ref, degt_ref, b_ref, w_ref, out_ref):
    dinv = _dinv_of(degt_ref[...])
    a = (a0_ref[0] + a1_ref[0] + g_ref[...]) * dinv + b_ref[...]
    r = jnp.maximum(a, 0.0)
    h = jnp.dot(r, w_ref[...], preferred_element_type=jnp.float32)
    out_ref[...] = h * dinv


def _tc_out_body(a0_ref, a1_ref, g_ref, degt_ref, b_ref, w_ref, b3_ref, out_ref):
    a = (a0_ref[0] + a1_ref[0] + g_ref[...]) * _dinv_of(degt_ref[...]) \
        + b_ref[...]
    r = jnp.maximum(a, 0.0)
    out_ref[...] = jnp.dot(r, w_ref[...],
                           preferred_element_type=jnp.float32) + b3_ref[...]


def _row_spec(width):
    return pl.BlockSpec((_RB, width), lambda i: (i, 0))


def _agg_spec(core):
    return pl.BlockSpec((1, _RB, H), lambda i, _c=core: (_c, i, 0))


def _full_spec(shape):
    return pl.BlockSpec(shape, lambda i: tuple(0 for _ in shape))


def _tc1(x, W1, degt):
    return pl.pallas_call(
        _tc1_body,
        grid=(_GRID,),
        in_specs=[_row_spec(D_IN), _full_spec((D_IN, H)), _row_spec(NW)],
        out_specs=_row_spec(H),
        out_shape=jax.ShapeDtypeStruct((N, H), jnp.float32),
    )(x, W1, degt)


def _tc_mid(agg, g, degt, b_row, W):
    return pl.pallas_call(
        _tc_mid_body,
        grid=(_GRID,),
        in_specs=[_agg_spec(0), _agg_spec(1), _row_spec(H), _row_spec(NW),
                  _full_spec((1, H)), _full_spec((H, H))],
        out_specs=_row_spec(H),
        out_shape=jax.ShapeDtypeStruct((N, H), jnp.float32),
    )(agg, agg, g, degt, b_row, W)


def _tc_out(agg, g, degt, b_row, W3, b3_row):
    return pl.pallas_call(
        _tc_out_body,
        grid=(_GRID,),
        in_specs=[_agg_spec(0), _agg_spec(1), _row_spec(H), _row_spec(NW),
                  _full_spec((1, H)), _full_spec((H, 7)), _full_spec((1, 7))],
        out_specs=_row_spec(7),
        out_shape=jax.ShapeDtypeStruct((N, 7), jnp.float32),
    )(agg, agg, g, degt, b_row, W3, b3_row)


@jax.jit
def _run(x, src, dst, W1, b1, W2, b2, W3, b3):
    src = src.astype(jnp.int32)
    dst2 = dst.astype(jnp.int32).reshape(NW, RPT)
    padv = N + jnp.arange(NPAD, dtype=jnp.int32)
    dstp = jnp.concatenate(
        [dst2, jnp.broadcast_to(padv, (NW, NPAD))], axis=1
    ).reshape(NW, ECH, EB)

    deg_parts = _make_sc_deg()(dstp)
    degt = deg_parts.T

    g1 = _tc1(x, W1, degt)

    agg_fn = _make_sc_agg()
    a1 = agg_fn(g1, src, dstp)
    g2 = _tc_mid(a1, g1, degt, b1.reshape(1, H), W2)
    a2 = agg_fn(g2, src, dstp)
    return _tc_out(a2, g2, degt, b2.reshape(1, H), W3, b3.reshape(1, 7))


def kernel(x, edge_index, W1, b1, W2, b2, W3, b3):
    return _run(x, edge_index[0], edge_index[1], W1, b1, W2, b2, W3, b3)


# speedup vs baseline: 1.3560x; 1.3560x over previous
import jax
import jax.numpy as jnp
from jax import lax
from jax.experimental import pallas as pl
from jax.experimental.pallas import tpu as pltpu
from jax.experimental.pallas import tpu_sc as plsc

N = 10000
E = 320000
D_IN = 128
H = 16
NCORES = 2
NSUB = 16
NW = NCORES * NSUB
EB = 80
ECH = 125
EPT = EB * ECH
RPS = N // NSUB

_mesh = plsc.VectorSubcoreMesh(core_axis_name="c", subcore_axis_name="s",
                               num_cores=NCORES, num_subcores=NSUB)

_sc_params = pltpu.CompilerParams(needs_layout_passes=False,
                                  use_tc_tiling_on_sc=False)


def _sc_deg_body(dst_hbm, deg_hbm, ldeg, dstb, isem):
    c = lax.axis_index("c")
    s = lax.axis_index("s")
    t = c * NSUB + s

    cp = pltpu.async_copy(dst_hbm.at[t], dstb, isem)

    zero16 = jnp.zeros((H,), jnp.float32)

    def _zero(i, _):
        ldeg[pl.ds(i * H, H)] = zero16
        return 0
    lax.fori_loop(0, N // H, _zero, 0)

    cp.wait()
    ones16 = jnp.ones((H,), jnp.float32)

    def _hist(r, _):
        for k in range(EB // H):
            idx = dstb[r, pl.ds(k * H, H)]
            plsc.addupdate_scatter(ldeg, [idx], ones16)
        return 0
    lax.fori_loop(0, ECH, _hist, 0)

    pltpu.sync_copy(ldeg, deg_hbm.at[t])


def _make_sc_deg():
    return pl.kernel(
        _sc_deg_body,
        out_type=jax.ShapeDtypeStruct((NW, N), jnp.float32),
        mesh=_mesh,
        scratch_types=[
            pltpu.VMEM((N,), jnp.float32),
            pltpu.VMEM((ECH, EB), jnp.int32),
            pltpu.SemaphoreType.DMA,
        ],
        compiler_params=_sc_params,
    )


GRP = 5
NG = ECH // GRP


def _sc_agg_kernel(g_hbm, src_hbm, dst_hbm, agg_hbm, acc, srcb, dstb,
                   bufs_a, bufs_b, zbuf, gsem_a, gsem_b, ssem_a, ssem_b,
                   isem):
    c = lax.axis_index("c")
    s = lax.axis_index("s")
    t = c * NSUB + s

    cp_src = pltpu.async_copy(src_hbm.at[t], srcb, isem)
    cp_dst = pltpu.async_copy(dst_hbm.at[t], dstb, isem)

    zero16 = jnp.zeros((H,), jnp.float32)

    def _zero(i, _):
        zbuf[i] = zero16
        return 0
    lax.fori_loop(0, RPS, _zero, 0)
    pltpu.sync_copy(zbuf, acc.at[pl.ds(RPS * s, RPS)])

    cp_src.wait()
    cp_dst.wait()
    plsc.subcore_barrier()

    def _drain(sem):
        pltpu.make_async_copy(g_hbm.at[pl.ds(0, EB)], bufs_a.at[0], sem).wait()

    def _gathers(grp, bufs, gsem):
        for k in range(GRP):
            pltpu.async_copy(g_hbm.at[srcb.at[grp * GRP + k]], bufs.at[k], gsem)

    def _scatters(grp, bufs, ssem):
        for k in range(GRP):
            pltpu.async_copy(bufs.at[k], acc.at[dstb.at[grp * GRP + k]], ssem,
                             add=True)

    _gathers(0, bufs_a, gsem_a)

    def _step(i, _):
        for _k in range(GRP):
            _drain(gsem_a)
        _scatters(2 * i, bufs_a, ssem_a)

        @pl.when(i > 0)
        def _():
            for _k in range(GRP):
                _drain(ssem_b)
        _gathers(2 * i + 1, bufs_b, gsem_b)

        for _k in range(GRP):
            _drain(gsem_b)
        _scatters(2 * i + 1, bufs_b, ssem_b)
        for _k in range(GRP):
            _drain(ssem_a)
        _gathers(2 * i + 2, bufs_a, gsem_a)
        return 0
    lax.fori_loop(0, (NG - 1) // 2, _step, 0)

    for _k in range(GRP):
        _drain(gsem_a)
    _scatters(NG - 1, bufs_a, ssem_a)
    for _k in range(GRP):
        _drain(ssem_b)
    for _k in range(GRP):
        _drain(ssem_a)

    plsc.subcore_barrier()
    pltpu.sync_copy(acc.at[pl.ds(RPS * s, RPS)],
                    agg_hbm.at[c].at[pl.ds(RPS * s, RPS)])


def _make_sc_agg():
    return pl.kernel(
        _sc_agg_kernel,
        out_type=jax.ShapeDtypeStruct((NCORES, N, H), jnp.float32),
        mesh=_mesh,
        scratch_types=[
            pltpu.VMEM_SHARED((N, H), jnp.float32),
            pltpu.VMEM((ECH, EB), jnp.int32),
            pltpu.VMEM((ECH, EB), jnp.int32),
            pltpu.VMEM((GRP, EB, H), jnp.float32),
            pltpu.VMEM((GRP, EB, H), jnp.float32),
            pltpu.VMEM((RPS, H), jnp.float32),
            pltpu.SemaphoreType.DMA,
            pltpu.SemaphoreType.DMA,
            pltpu.SemaphoreType.DMA,
            pltpu.SemaphoreType.DMA,
            pltpu.SemaphoreType.DMA,
        ],
        compiler_params=_sc_params,
    )


_RB = 5000
_GRID = N // _RB


def _dinv_of(degt_block):
    return lax.rsqrt(jnp.sum(degt_block, axis=1, keepdims=True) + 1.0)


def _tc1_body(x_ref, w_ref, degt_ref, g_ref):
    h = jnp.dot(x_ref[...], w_ref[...], preferred_element_type=jnp.float32)
    g_ref[...] = h * _dinv_of(degt_ref[...])


def _tc_mid_body(a0_ref, a1_ref, g_ref, degt_ref, b_ref, w_ref, out_ref):
    dinv = _dinv_of(degt_ref[...])
    a = (a0_ref[0] + a1_ref[0] + g_ref[...]) * dinv + b_ref[...]
    r = jnp.maximum(a, 0.0)
    h = jnp.dot(r, w_ref[...], preferred_element_type=jnp.float32)
    out_ref[...] = h * dinv


def _tc_out_body(a0_ref, a1_ref, g_ref, degt_ref, b_ref, w_ref, b3_ref, out_ref):
    a = (a0_ref[0] + a1_ref[0] + g_ref[...]) * _dinv_of(degt_ref[...]) \
        + b_ref[...]
    r = jnp.maximum(a, 0.0)
    out_ref[...] = jnp.dot(r, w_ref[...],
                           preferred_element_type=jnp.float32) + b3_ref[...]


def _row_spec(width):
    return pl.BlockSpec((_RB, width), lambda i: (i, 0))


def _agg_spec(core):
    return pl.BlockSpec((1, _RB, H), lambda i, _c=core: (_c, i, 0))


def _full_spec(shape):
    return pl.BlockSpec(shape, lambda i: tuple(0 for _ in shape))


def _tc1(x, W1, degt):
    return pl.pallas_call(
        _tc1_body,
        grid=(_GRID,),
        in_specs=[_row_spec(D_IN), _full_spec((D_IN, H)), _row_spec(NW)],
        out_specs=_row_spec(H),
        out_shape=jax.ShapeDtypeStruct((N, H), jnp.float32),
    )(x, W1, degt)


def _tc_mid(agg, g, degt, b_row, W):
    return pl.pallas_call(
        _tc_mid_body,
        grid=(_GRID,),
        in_specs=[_agg_spec(0), _agg_spec(1), _row_spec(H), _row_spec(NW),
                  _full_spec((1, H)), _full_spec((H, H))],
        out_specs=_row_spec(H),
        out_shape=jax.ShapeDtypeStruct((N, H), jnp.float32),
    )(agg, agg, g, degt, b_row, W)


def _tc_out(agg, g, degt, b_row, W3, b3_row):
    return pl.pallas_call(
        _tc_out_body,
        grid=(_GRID,),
        in_specs=[_agg_spec(0), _agg_spec(1), _row_spec(H), _row_spec(NW),
                  _full_spec((1, H)), _full_spec((H, 7)), _full_spec((1, 7))],
        out_specs=_row_spec(7),
        out_shape=jax.ShapeDtypeStruct((N, 7), jnp.float32),
    )(agg, agg, g, degt, b_row, W3, b3_row)


@jax.jit
def _run(x, src, dst, W1, b1, W2, b2, W3, b3):
    srcp = src.astype(jnp.int32).reshape(NW, ECH, EB)
    dstp = dst.astype(jnp.int32).reshape(NW, ECH, EB)

    deg_parts = _make_sc_deg()(dstp)
    degt = deg_parts.T

    g1 = _tc1(x, W1, degt)

    agg_fn = _make_sc_agg()
    a1 = agg_fn(g1, srcp, dstp)
    g2 = _tc_mid(a1, g1, degt, b1.reshape(1, H), W2)
    a2 = agg_fn(g2, srcp, dstp)
    return _tc_out(a2, g2, degt, b2.reshape(1, H), W3, b3.reshape(1, 7))


def kernel(x, edge_index, W1, b1, W2, b2, W3, b3):
    return _run(x, edge_index[0], edge_index[1], W1, b1, W2, b2, W3, b3)
